# baseline (speedup 1.0000x reference)
from functools import partial

import jax
import jax.numpy as jnp
import numpy as np
from jax.experimental import pallas as pl
from jax.experimental.pallas import tpu as pltpu
from jax.sharding import Mesh, PartitionSpec as P

N = 4096
D = 128
FC1 = 64
BM = 512


def _agg_kernel(V_ref, adj_ref, w1_ref, w2_ref, w3_ref, bg_ref, out_ref,
                h1_ref, h2_ref, h3_ref, zsum_ref, *, num_blocks):
    i = pl.program_id(0)

    @pl.when(i == 0)
    def _compute_h():
        vblk = V_ref[:, :]
        h1_ref[:, :] = jnp.dot(
            vblk, w1_ref[:, :],
            preferred_element_type=jnp.float32).astype(jnp.bfloat16)
        h2_ref[:, :] = jnp.dot(
            vblk, w2_ref[:, :],
            preferred_element_type=jnp.float32).astype(jnp.bfloat16)
        h3_ref[:, :] = jnp.dot(
            vblk, w3_ref[:, :],
            preferred_element_type=jnp.float32).astype(jnp.bfloat16)

    a = adj_ref[:, :]
    m1 = (a == 1).astype(jnp.bfloat16)
    m2 = (a == 2).astype(jnp.bfloat16)
    m3 = (a == 3).astype(jnp.bfloat16)
    o = (jnp.dot(m1, h1_ref[:, :], preferred_element_type=jnp.float32)
         + jnp.dot(m2, h2_ref[:, :], preferred_element_type=jnp.float32)
         + jnp.dot(m3, h3_ref[:, :], preferred_element_type=jnp.float32))
    z = jnp.maximum(o + bg_ref[:, :], 0.0)
    part = jnp.sum(z, axis=0, keepdims=True)

    @pl.when(i == 0)
    def _init():
        zsum_ref[:, :] = part

    @pl.when(i > 0)
    def _accum():
        zsum_ref[:, :] += part

    @pl.when(i == num_blocks - 1)
    def _emit():
        out_ref[:, :] = zsum_ref[:, :]


def _head_kernel(z_ref, fc0w_ref, fc0b_ref, fc1w_ref, fc1b_ref, out_ref):
    h0 = jax.lax.dot_general(
        z_ref[:, :], fc0w_ref[:, :], (((1,), (1,)), ((), ())),
        preferred_element_type=jnp.float32) + fc0b_ref[:, :]
    h0 = jnp.maximum(h0, 0.0)
    y = jnp.sum(h0 * fc1w_ref[:, :] + fc1b_ref[:, :])
    out_ref[:, :] = jnp.full((1, 1), jax.nn.sigmoid(y), jnp.float32)


def _local_zsum(V, adj_loc, w1, w2, w3, bg2):
    rows = adj_loc.shape[0]
    nb = rows // BM
    return pl.pallas_call(
        partial(_agg_kernel, num_blocks=nb),
        grid=(nb,),
        in_specs=[
            pl.BlockSpec((N, D), lambda i: (0, 0)),
            pl.BlockSpec((BM, N), lambda i: (i, 0)),
            pl.BlockSpec((D, D), lambda i: (0, 0)),
            pl.BlockSpec((D, D), lambda i: (0, 0)),
            pl.BlockSpec((D, D), lambda i: (0, 0)),
            pl.BlockSpec((1, D), lambda i: (0, 0)),
        ],
        out_specs=pl.BlockSpec((1, D), lambda i: (0, 0)),
        out_shape=jax.ShapeDtypeStruct((1, D), jnp.float32),
        scratch_shapes=[
            pltpu.VMEM((N, D), jnp.bfloat16),
            pltpu.VMEM((N, D), jnp.bfloat16),
            pltpu.VMEM((N, D), jnp.bfloat16),
            pltpu.VMEM((1, D), jnp.float32),
        ],
    )(V, adj_loc, w1, w2, w3, bg2)


def _head(z, fc0_w, fc0b2, fc1_w, fc1b2):
    return pl.pallas_call(
        _head_kernel,
        in_specs=[
            pl.BlockSpec((1, D), lambda: (0, 0)),
            pl.BlockSpec((FC1, D), lambda: (0, 0)),
            pl.BlockSpec((1, FC1), lambda: (0, 0)),
            pl.BlockSpec((1, FC1), lambda: (0, 0)),
            pl.BlockSpec((1, FC1), lambda: (0, 0)),
        ],
        out_specs=pl.BlockSpec((1, 1), lambda: (0, 0)),
        out_shape=jax.ShapeDtypeStruct((1, 1), jnp.float32),
    )(z, fc0_w, fc0b2, fc1_w, fc1b2)


def kernel(V, adj, w1, w2, w3, bg, fc0_w, fc0_b, fc1_w, fc1_b):
    bg2 = bg.reshape(1, D)
    fc0b2 = fc0_b.reshape(1, FC1)
    fc1b2 = jnp.broadcast_to(fc1_b.reshape(1, 1) / FC1, (1, FC1))
    mesh = Mesh(np.array(jax.devices()[:2]), ("x",))

    @partial(
        jax.shard_map, mesh=mesh,
        in_specs=(P(), P("x", None), P(), P(), P(), P(),
                  P(), P(), P(), P()),
        out_specs=P(),
        check_vma=False,
    )
    def _sharded(V, adj_loc, w1, w2, w3, bg2, fc0_w, fc0b2, fc1_w, fc1b2):
        zloc = _local_zsum(V, adj_loc, w1, w2, w3, bg2)
        z = jax.lax.psum(zloc, "x")
        return _head(z, fc0_w, fc0b2, fc1_w, fc1b2)

    out = _sharded(V, adj, w1, w2, w3, bg2, fc0_w, fc0b2, fc1_w, fc1b2)
    return out.reshape(1)


# speedup vs baseline: 17.0146x; 17.0146x over previous
import jax
import jax.numpy as jnp
from jax.experimental import pallas as pl
from jax.experimental.pallas import tpu as pltpu

N = 4096
D = 128
FC1 = 64
BM = 512
IB = N // BM
NBUF = 3
NSUB = 4
SUBR = BM // NSUB


def _gcn_kernel(V_ref, adj_hbm, w1_ref, w2_ref, w3_ref, bg_ref,
                fc0w_ref, fc0b_ref, fc1w_ref, fc1b_ref, out_ref,
                abuf, h1_ref, h2_ref, h3_ref, sem):

    def copies(blk, slot):
        return [
            pltpu.make_async_copy(
                adj_hbm.at[pl.ds(blk * BM + q * SUBR, SUBR), :],
                abuf.at[slot, pl.ds(q * SUBR, SUBR), :],
                sem.at[slot, q])
            for q in range(NSUB)
        ]

    for b in range(NBUF):
        for c in copies(b, b):
            c.start()

    vblk = V_ref[:, :]
    h1_ref[:, :] = jnp.dot(
        vblk, w1_ref[:, :],
        preferred_element_type=jnp.float32).astype(jnp.bfloat16)
    h2_ref[:, :] = jnp.dot(
        vblk, w2_ref[:, :],
        preferred_element_type=jnp.float32).astype(jnp.bfloat16)
    h3_ref[:, :] = jnp.dot(
        vblk, w3_ref[:, :],
        preferred_element_type=jnp.float32).astype(jnp.bfloat16)

    def body(k, zacc):
        slot = jax.lax.rem(k, NBUF)
        for c in copies(k, slot):
            c.wait()
        a = abuf[slot]
        m1 = (a == 1).astype(jnp.bfloat16)
        m2 = (a == 2).astype(jnp.bfloat16)
        m3 = (a == 3).astype(jnp.bfloat16)
        o = (jnp.dot(m1, h1_ref[:, :], preferred_element_type=jnp.float32)
             + jnp.dot(m2, h2_ref[:, :], preferred_element_type=jnp.float32)
             + jnp.dot(m3, h3_ref[:, :], preferred_element_type=jnp.float32))

        @pl.when(k + NBUF < IB)
        def _prefetch():
            for c in copies(k + NBUF, slot):
                c.start()

        z = jnp.maximum(o + bg_ref[:, :], 0.0)
        return zacc + jnp.sum(z, axis=0, keepdims=True)

    zs = jax.lax.fori_loop(0, IB, body, jnp.zeros((1, D), jnp.float32))

    h0 = jax.lax.dot_general(
        zs, fc0w_ref[:, :], (((1,), (1,)), ((), ())),
        preferred_element_type=jnp.float32) + fc0b_ref[:, :]
    h0 = jnp.maximum(h0, 0.0)
    y = jnp.sum(h0 * fc1w_ref[:, :] + fc1b_ref[:, :])
    out_ref[:, :] = jnp.full((1, 1), jax.nn.sigmoid(y), jnp.float32)


def kernel(V, adj, w1, w2, w3, bg, fc0_w, fc0_b, fc1_w, fc1_b):
    bg2 = bg.reshape(1, D)
    fc0b2 = fc0_b.reshape(1, FC1)
    fc1b2 = jnp.broadcast_to(fc1_b.reshape(1, 1) / FC1, (1, FC1))
    out = pl.pallas_call(
        _gcn_kernel,
        in_specs=[
            pl.BlockSpec((N, D), lambda: (0, 0)),
            pl.BlockSpec(memory_space=pl.ANY),
            pl.BlockSpec((D, D), lambda: (0, 0)),
            pl.BlockSpec((D, D), lambda: (0, 0)),
            pl.BlockSpec((D, D), lambda: (0, 0)),
            pl.BlockSpec((1, D), lambda: (0, 0)),
            pl.BlockSpec((FC1, D), lambda: (0, 0)),
            pl.BlockSpec((1, FC1), lambda: (0, 0)),
            pl.BlockSpec((1, FC1), lambda: (0, 0)),
            pl.BlockSpec((1, FC1), lambda: (0, 0)),
        ],
        out_specs=pl.BlockSpec((1, 1), lambda: (0, 0)),
        out_shape=jax.ShapeDtypeStruct((1, 1), jnp.float32),
        scratch_shapes=[
            pltpu.VMEM((NBUF, BM, N), jnp.int32),
            pltpu.VMEM((N, D), jnp.bfloat16),
            pltpu.VMEM((N, D), jnp.bfloat16),
            pltpu.VMEM((N, D), jnp.bfloat16),
            pltpu.SemaphoreType.DMA((NBUF, NSUB)),
        ],
        compiler_params=pltpu.CompilerParams(
            vmem_limit_bytes=100 * 1024 * 1024),
    )(V, adj, w1, w2, w3, bg2, fc0_w, fc0b2, fc1_w, fc1b2)
    return out.reshape(1)


# speedup vs baseline: 17.4443x; 1.0253x over previous
import jax
import jax.numpy as jnp
from jax.experimental import pallas as pl
from jax.experimental.pallas import tpu as pltpu

N = 4096
D = 128
FC1 = 64
BM = 512
IB = N // BM
NBUF = 3
NSUB = 4
SUBR = BM // NSUB


def _gcn_kernel(V_ref, adj_hbm, w1_ref, w2_ref, w3_ref, bg_ref,
                fc0w_ref, fc0b_ref, fc1w_ref, fc1b_ref, out_ref,
                abuf, hcat_ref, sem):

    def copies(blk, slot):
        return [
            pltpu.make_async_copy(
                adj_hbm.at[pl.ds(blk * BM + q * SUBR, SUBR), :],
                abuf.at[slot, pl.ds(q * SUBR, SUBR), :],
                sem.at[slot, q])
            for q in range(NSUB)
        ]

    for b in range(NBUF):
        for c in copies(b, b):
            c.start()

    vblk = V_ref[:, :]
    hcat_ref[pl.ds(0, N), :] = jnp.dot(
        vblk, w1_ref[:, :],
        preferred_element_type=jnp.float32).astype(jnp.bfloat16)
    hcat_ref[pl.ds(N, N), :] = jnp.dot(
        vblk, w2_ref[:, :],
        preferred_element_type=jnp.float32).astype(jnp.bfloat16)
    hcat_ref[pl.ds(2 * N, N), :] = jnp.dot(
        vblk, w3_ref[:, :],
        preferred_element_type=jnp.float32).astype(jnp.bfloat16)

    def body(k, zacc):
        slot = jax.lax.rem(k, NBUF)
        for c in copies(k, slot):
            c.wait()
        a = abuf[slot]
        mcat = jnp.concatenate(
            [(a == 1).astype(jnp.bfloat16),
             (a == 2).astype(jnp.bfloat16),
             (a == 3).astype(jnp.bfloat16)], axis=1)
        o = jnp.dot(mcat, hcat_ref[:, :], preferred_element_type=jnp.float32)

        @pl.when(k + NBUF < IB)
        def _prefetch():
            for c in copies(k + NBUF, slot):
                c.start()

        z = jnp.maximum(o + bg_ref[:, :], 0.0)
        return zacc + jnp.sum(z, axis=0, keepdims=True)

    zs = jax.lax.fori_loop(0, IB, body, jnp.zeros((1, D), jnp.float32))

    h0 = jax.lax.dot_general(
        zs, fc0w_ref[:, :], (((1,), (1,)), ((), ())),
        preferred_element_type=jnp.float32) + fc0b_ref[:, :]
    h0 = jnp.maximum(h0, 0.0)
    y = jnp.sum(h0 * fc1w_ref[:, :] + fc1b_ref[:, :])
    out_ref[:, :] = jnp.full((1, 1), jax.nn.sigmoid(y), jnp.float32)


def kernel(V, adj, w1, w2, w3, bg, fc0_w, fc0_b, fc1_w, fc1_b):
    bg2 = bg.reshape(1, D)
    fc0b2 = fc0_b.reshape(1, FC1)
    fc1b2 = jnp.broadcast_to(fc1_b.reshape(1, 1) / FC1, (1, FC1))
    out = pl.pallas_call(
        _gcn_kernel,
        in_specs=[
            pl.BlockSpec((N, D), lambda: (0, 0)),
            pl.BlockSpec(memory_space=pl.ANY),
            pl.BlockSpec((D, D), lambda: (0, 0)),
            pl.BlockSpec((D, D), lambda: (0, 0)),
            pl.BlockSpec((D, D), lambda: (0, 0)),
            pl.BlockSpec((1, D), lambda: (0, 0)),
            pl.BlockSpec((FC1, D), lambda: (0, 0)),
            pl.BlockSpec((1, FC1), lambda: (0, 0)),
            pl.BlockSpec((1, FC1), lambda: (0, 0)),
            pl.BlockSpec((1, FC1), lambda: (0, 0)),
        ],
        out_specs=pl.BlockSpec((1, 1), lambda: (0, 0)),
        out_shape=jax.ShapeDtypeStruct((1, 1), jnp.float32),
        scratch_shapes=[
            pltpu.VMEM((NBUF, BM, N), jnp.int32),
            pltpu.VMEM((3 * N, D), jnp.bfloat16),
            pltpu.SemaphoreType.DMA((NBUF, NSUB)),
        ],
        compiler_params=pltpu.CompilerParams(
            vmem_limit_bytes=100 * 1024 * 1024),
    )(V, adj, w1, w2, w3, bg2, fc0_w, fc0b2, fc1_w, fc1b2)
    return out.reshape(1)
